# baseline (speedup 1.0000x reference)
import functools

import jax
import jax.numpy as jnp
from jax import lax
from jax.experimental import pallas as pl
from jax.experimental.pallas import tpu as pltpu
from jax.experimental.pallas import tpu_sc as plsc

B = 16384
D = 64
NT = 6
CHUNK = 256
GRP0 = 4


def _mlp_body(lat_ref, lon_ref, w1_ref, b1_ref, w2_ref, b2_ref, o_ref):
    h = jnp.maximum(
        lat_ref[...] * w1_ref[0:1, :] + lon_ref[...] * w1_ref[1:2, :]
        + b1_ref[...],
        0.0,
    )
    o_ref[...] = (
        jnp.dot(h, w2_ref[...], preferred_element_type=jnp.float32)
        + b2_ref[...]
    )


def _mlp(latitude, longitude, W1, b1, W2, b2):
    return pl.pallas_call(
        _mlp_body,
        out_shape=jax.ShapeDtypeStruct((B, D), jnp.float32),
    )(
        latitude.reshape(B, 1),
        longitude.reshape(B, 1),
        W1,
        b1.reshape(1, 32),
        W2,
        b2.reshape(1, D),
    )


def _sc_embed(h_i, s_i, m_i, hr_i, cmod_i, cmak_i, g,
              h_t, s_t, m_t, hr_t, cmod_t, cmak_t):
    info = plsc.get_sparse_core_info()
    NC, NS = info.num_cores, info.num_subcores
    NW = NC * NS
    b_per_w = B // NW
    n_sub = b_per_w // CHUNK
    n_tasks = n_sub * 2

    mesh = plsc.VectorSubcoreMesh(core_axis_name="c", subcore_axis_name="s")

    @functools.partial(
        pl.kernel,
        mesh=mesh,
        out_type=jax.ShapeDtypeStruct((B, (NT + 1) * D), jnp.float32),
        scratch_types=[
            pltpu.VMEM((NT, b_per_w), jnp.int32),
            pltpu.VMEM((NT + 1, CHUNK, D), jnp.float32),
            pltpu.SemaphoreType.DMA,
            pltpu.SemaphoreType.DMA,
            pltpu.SemaphoreType.DMA,
            pltpu.SemaphoreType.DMA,
        ],
        compiler_params=pltpu.CompilerParams(use_tc_tiling_on_sc=False),
    )
    def k(h_ref, s_ref, m_ref, hr_ref, cmod_ref, cmak_ref, g_ref,
          ht_ref, st_ref, mt_ref, hrt_ref, cmodt_ref, cmakt_ref,
          out_ref, idx_v, bufs, sem_g0, sem_g1, sem_w0, sem_w1):
        wid = lax.axis_index("s") * NC + lax.axis_index("c")
        base = wid * b_per_w
        idx_hbm = [h_ref, s_ref, m_ref, hr_ref, cmod_ref, cmak_ref]
        tbls = [ht_ref, st_ref, mt_ref, hrt_ref, cmodt_ref, cmakt_ref]
        sem_g = [sem_g0, sem_g1]
        sem_w = [sem_w0, sem_w1]
        for t in range(NT):
            pltpu.sync_copy(idx_hbm[t].at[pl.ds(base, b_per_w)], idx_v.at[t])

        def task_slots(grp):
            if grp == 0:
                return [(t, t) for t in range(GRP0)]
            return [(t, t) for t in range(GRP0, NT)] + [(NT, NT)]

        def fire_task(i):
            c, grp = i // 2, i % 2
            off = base + c * CHUNK
            ds = []
            for t, slot in task_slots(grp):
                if t < NT:
                    ds.append(pltpu.async_copy(
                        tbls[t].at[idx_v.at[t, pl.ds(c * CHUNK, CHUNK)]],
                        bufs.at[slot], sem_g[grp]))
                else:
                    ds.append(pltpu.async_copy(
                        g_ref.at[pl.ds(off, CHUNK)], bufs.at[slot],
                        sem_g[grp]))
            return ds

        def fire_writes(i):
            c, grp = i // 2, i % 2
            off = base + c * CHUNK
            ds = []
            for t, slot in task_slots(grp):
                ds.append(pltpu.async_copy(
                    bufs.at[slot],
                    out_ref.at[pl.ds(off, CHUNK), pl.ds(t * D, D)],
                    sem_w[grp]))
            return ds

        writes = {0: [], 1: []}
        gath = {0: [], 1: []}
        for i in range(n_tasks + 1):
            if i < n_tasks:
                s = i % 2
                for wdesc in writes[s]:
                    wdesc.wait()
                writes[s] = []
                gath[s] = fire_task(i)
            if i >= 1:
                sj = (i - 1) % 2
                for gd in gath[sj]:
                    gd.wait()
                gath[sj] = []
                writes[sj] = fire_writes(i - 1)
        for s in (0, 1):
            for wdesc in writes[s]:
                wdesc.wait()

    return k(h_i, s_i, m_i, hr_i, cmod_i, cmak_i, g,
             h_t, s_t, m_t, hr_t, cmod_t, cmak_t)


def kernel(habitat, substrate, month, hour, camera_model, camera_maker,
           latitude, longitude,
           habitat_table, substrate_table, month_table, hour_table,
           camera_model_table, camera_maker_table, W1, b1, W2, b2):
    g = _mlp(latitude, longitude, W1, b1, W2, b2)
    idx = [x.astype(jnp.int32) for x in
           (habitat, substrate, month, hour, camera_model, camera_maker)]
    return _sc_embed(*idx, g,
                     habitat_table, substrate_table, month_table, hour_table,
                     camera_model_table, camera_maker_table)


# speedup vs baseline: 1.0116x; 1.0116x over previous
import functools

import jax
import jax.numpy as jnp
from jax import lax
from jax.experimental import pallas as pl
from jax.experimental.pallas import tpu as pltpu
from jax.experimental.pallas import tpu_sc as plsc

B = 16384
D = 64
NT = 6
CHUNK = 256
GRP0 = 4


def _mlp_body(lat_ref, lon_ref, w1_ref, b1_ref, w2_ref, b2_ref, o_ref):
    h = jnp.maximum(
        lat_ref[...] * w1_ref[0:1, :] + lon_ref[...] * w1_ref[1:2, :]
        + b1_ref[...],
        0.0,
    )
    o_ref[...] = (
        jnp.dot(h, w2_ref[...], preferred_element_type=jnp.float32)
        + b2_ref[...]
    )


def _mlp(latitude, longitude, W1, b1, W2, b2):
    return pl.pallas_call(
        _mlp_body,
        out_shape=jax.ShapeDtypeStruct((B, D), jnp.float32),
    )(
        latitude.reshape(B, 1),
        longitude.reshape(B, 1),
        W1,
        b1.reshape(1, 32),
        W2,
        b2.reshape(1, D),
    )


def _sc_embed(h_i, s_i, m_i, hr_i, cmod_i, cmak_i, g,
              h_t, s_t, m_t, hr_t, cmod_t, cmak_t):
    info = plsc.get_sparse_core_info()
    NC, NS = info.num_cores, info.num_subcores
    NW = NC * NS
    b_per_w = B // NW
    n_sub = b_per_w // CHUNK
    n_tasks = n_sub * 2

    mesh = plsc.VectorSubcoreMesh(core_axis_name="c", subcore_axis_name="s")

    @functools.partial(
        pl.kernel,
        mesh=mesh,
        out_type=jax.ShapeDtypeStruct((B, (NT + 1) * D), jnp.float32),
        scratch_types=[
            pltpu.VMEM((NT, b_per_w), jnp.int32),
            pltpu.VMEM((NT + 1, CHUNK, D), jnp.float32),
            pltpu.SemaphoreType.DMA,
            pltpu.SemaphoreType.DMA,
            pltpu.SemaphoreType.DMA,
            pltpu.SemaphoreType.DMA,
        ],
        compiler_params=pltpu.CompilerParams(use_tc_tiling_on_sc=False),
    )
    def k(h_ref, s_ref, m_ref, hr_ref, cmod_ref, cmak_ref, g_ref,
          ht_ref, st_ref, mt_ref, hrt_ref, cmodt_ref, cmakt_ref,
          out_ref, idx_v, bufs, sem_g0, sem_g1, sem_w0, sem_w1):
        wid = lax.axis_index("s") * NC + lax.axis_index("c")
        base = wid * b_per_w
        idx_hbm = [h_ref, s_ref, m_ref, hr_ref, cmod_ref, cmak_ref]
        tbls = [ht_ref, st_ref, mt_ref, hrt_ref, cmodt_ref, cmakt_ref]
        sem_g = [sem_g0, sem_g1]
        sem_w = [sem_w0, sem_w1]
        idx_copies = [
            pltpu.async_copy(idx_hbm[t].at[pl.ds(base, b_per_w)],
                             idx_v.at[t], sem_g0)
            for t in range(NT)
        ]
        for d in idx_copies:
            d.wait()

        def task_slots(grp):
            if grp == 0:
                return [(t, t) for t in range(GRP0)]
            return [(t, t) for t in range(GRP0, NT)] + [(NT, NT)]

        def fire_task(i):
            c, grp = i // 2, i % 2
            off = base + c * CHUNK
            ds = []
            for t, slot in task_slots(grp):
                if t < NT:
                    ds.append(pltpu.async_copy(
                        tbls[t].at[idx_v.at[t, pl.ds(c * CHUNK, CHUNK)]],
                        bufs.at[slot], sem_g[grp]))
                else:
                    ds.append(pltpu.async_copy(
                        g_ref.at[pl.ds(off, CHUNK)], bufs.at[slot],
                        sem_g[grp]))
            return ds

        def fire_writes(i):
            c, grp = i // 2, i % 2
            off = base + c * CHUNK
            ds = []
            for t, slot in task_slots(grp):
                ds.append(pltpu.async_copy(
                    bufs.at[slot],
                    out_ref.at[pl.ds(off, CHUNK), pl.ds(t * D, D)],
                    sem_w[grp]))
            return ds

        writes = {0: [], 1: []}
        gath = {0: [], 1: []}
        for i in range(n_tasks + 1):
            if i < n_tasks:
                s = i % 2
                for wdesc in writes[s]:
                    wdesc.wait()
                writes[s] = []
                gath[s] = fire_task(i)
            if i >= 1:
                sj = (i - 1) % 2
                for gd in gath[sj]:
                    gd.wait()
                gath[sj] = []
                writes[sj] = fire_writes(i - 1)
        for s in (0, 1):
            for wdesc in writes[s]:
                wdesc.wait()

    return k(h_i, s_i, m_i, hr_i, cmod_i, cmak_i, g,
             h_t, s_t, m_t, hr_t, cmod_t, cmak_t)


def kernel(habitat, substrate, month, hour, camera_model, camera_maker,
           latitude, longitude,
           habitat_table, substrate_table, month_table, hour_table,
           camera_model_table, camera_maker_table, W1, b1, W2, b2):
    g = _mlp(latitude, longitude, W1, b1, W2, b2)
    idx = [x.astype(jnp.int32) for x in
           (habitat, substrate, month, hour, camera_model, camera_maker)]
    return _sc_embed(*idx, g,
                     habitat_table, substrate_table, month_table, hour_table,
                     camera_model_table, camera_maker_table)


# speedup vs baseline: 1.0594x; 1.0473x over previous
import functools

import jax
import jax.numpy as jnp
from jax import lax
from jax.experimental import pallas as pl
from jax.experimental.pallas import tpu as pltpu
from jax.experimental.pallas import tpu_sc as plsc

B = 16384
D = 64
NT = 6
CHUNK = 256
GRP0 = 3
BLK = 1024


def _sc_gather(h_i, s_i, m_i, hr_i, cmod_i, cmak_i,
               h_t, s_t, m_t, hr_t, cmod_t, cmak_t):
    info = plsc.get_sparse_core_info()
    NC, NS = info.num_cores, info.num_subcores
    NW = NC * NS
    b_per_w = B // NW
    n_sub = b_per_w // CHUNK
    n_tasks = n_sub * 2

    mesh = plsc.VectorSubcoreMesh(core_axis_name="c", subcore_axis_name="s")

    @functools.partial(
        pl.kernel,
        mesh=mesh,
        out_type=[jax.ShapeDtypeStruct((B, 2 * D), jnp.float32)
                  for _ in range(NT // 2)],
        scratch_types=[
            pltpu.VMEM((NT, b_per_w), jnp.int32),
            pltpu.VMEM((NT, CHUNK, D), jnp.float32),
            pltpu.SemaphoreType.DMA,
            pltpu.SemaphoreType.DMA,
            pltpu.SemaphoreType.DMA,
            pltpu.SemaphoreType.DMA,
        ],
        compiler_params=pltpu.CompilerParams(use_tc_tiling_on_sc=False),
    )
    def k(h_ref, s_ref, m_ref, hr_ref, cmod_ref, cmak_ref,
          ht_ref, st_ref, mt_ref, hrt_ref, cmodt_ref, cmakt_ref,
          o0, o1, o2, idx_v, bufs, sem_g0, sem_g1, sem_w0, sem_w1):
        wid = lax.axis_index("s") * NC + lax.axis_index("c")
        base = wid * b_per_w
        idx_hbm = [h_ref, s_ref, m_ref, hr_ref, cmod_ref, cmak_ref]
        tbls = [ht_ref, st_ref, mt_ref, hrt_ref, cmodt_ref, cmakt_ref]
        outs = [o0, o1, o2]
        sem_g = [sem_g0, sem_g1]
        sem_w = [sem_w0, sem_w1]
        idx_copies = [
            pltpu.async_copy(idx_hbm[t].at[pl.ds(base, b_per_w)],
                             idx_v.at[t], sem_g0)
            for t in range(NT)
        ]
        for d in idx_copies:
            d.wait()

        def group_tables(grp):
            return range(GRP0) if grp == 0 else range(GRP0, NT)

        def fire_task(i):
            c, grp = i // 2, i % 2
            ds = []
            for t in group_tables(grp):
                ds.append(pltpu.async_copy(
                    tbls[t].at[idx_v.at[t, pl.ds(c * CHUNK, CHUNK)]],
                    bufs.at[t], sem_g[grp]))
            return ds

        def fire_writes(i):
            c, grp = i // 2, i % 2
            off = base + c * CHUNK
            ds = []
            for t in group_tables(grp):
                ds.append(pltpu.async_copy(
                    bufs.at[t],
                    outs[t // 2].at[pl.ds(off, CHUNK),
                                    pl.ds((t % 2) * D, D)],
                    sem_w[grp]))
            return ds

        writes = {0: [], 1: []}
        gath = {0: [], 1: []}
        for i in range(n_tasks + 1):
            if i < n_tasks:
                s = i % 2
                for wdesc in writes[s]:
                    wdesc.wait()
                writes[s] = []
                gath[s] = fire_task(i)
            if i >= 1:
                sj = (i - 1) % 2
                for gd in gath[sj]:
                    gd.wait()
                gath[sj] = []
                writes[sj] = fire_writes(i - 1)
        for s in (0, 1):
            for wdesc in writes[s]:
                wdesc.wait()

    return k(h_i, s_i, m_i, hr_i, cmod_i, cmak_i,
             h_t, s_t, m_t, hr_t, cmod_t, cmak_t)


def _asm_body(p0, p1, p2, lat_ref, lon_ref,
              w1_ref, b1_ref, w2_ref, b2_ref, out_ref):
    for t, p in enumerate((p0, p1, p2)):
        out_ref[:, t * 2 * D:(t + 1) * 2 * D] = p[...]
    h = jnp.maximum(
        lat_ref[...] * w1_ref[0:1, :] + lon_ref[...] * w1_ref[1:2, :]
        + b1_ref[...],
        0.0,
    )
    out_ref[:, NT * D:] = (
        jnp.dot(h, w2_ref[...], preferred_element_type=jnp.float32)
        + b2_ref[...]
    )


def _assemble(planes, latitude, longitude, W1, b1, W2, b2):
    plane_spec = pl.BlockSpec((BLK, 2 * D), lambda i: (i, 0))
    col_spec = pl.BlockSpec((BLK, 1), lambda i: (i, 0))
    w1_spec = pl.BlockSpec((2, 32), lambda i: (0, 0))
    b1_spec = pl.BlockSpec((1, 32), lambda i: (0, 0))
    w2_spec = pl.BlockSpec((32, D), lambda i: (0, 0))
    b2_spec = pl.BlockSpec((1, D), lambda i: (0, 0))
    return pl.pallas_call(
        _asm_body,
        grid=(B // BLK,),
        in_specs=[plane_spec] * 3 + [col_spec, col_spec,
                                     w1_spec, b1_spec, w2_spec, b2_spec],
        out_specs=pl.BlockSpec((BLK, (NT + 1) * D), lambda i: (i, 0)),
        out_shape=jax.ShapeDtypeStruct((B, (NT + 1) * D), jnp.float32),
    )(*planes,
      latitude.reshape(B, 1), longitude.reshape(B, 1),
      W1, b1.reshape(1, 32), W2, b2.reshape(1, D))


def kernel(habitat, substrate, month, hour, camera_model, camera_maker,
           latitude, longitude,
           habitat_table, substrate_table, month_table, hour_table,
           camera_model_table, camera_maker_table, W1, b1, W2, b2):
    idx = [x.astype(jnp.int32) for x in
           (habitat, substrate, month, hour, camera_model, camera_maker)]
    planes = _sc_gather(*idx,
                        habitat_table, substrate_table, month_table,
                        hour_table, camera_model_table, camera_maker_table)
    return _assemble(planes, latitude, longitude, W1, b1, W2, b2)


# speedup vs baseline: 1.0841x; 1.0233x over previous
import functools

import jax
import jax.numpy as jnp
from jax import lax
from jax.experimental import pallas as pl
from jax.experimental.pallas import tpu as pltpu
from jax.experimental.pallas import tpu_sc as plsc

B = 16384
D = 64
NT = 6
CHUNK = 128
GRP0 = 3
BLK = 1024


def _sc_gather(h_i, s_i, m_i, hr_i, cmod_i, cmak_i,
               h_t, s_t, m_t, hr_t, cmod_t, cmak_t):
    info = plsc.get_sparse_core_info()
    NC, NS = info.num_cores, info.num_subcores
    NW = NC * NS
    b_per_w = B // NW
    n_sub = b_per_w // CHUNK
    n_tasks = n_sub * 2

    mesh = plsc.VectorSubcoreMesh(core_axis_name="c", subcore_axis_name="s")

    @functools.partial(
        pl.kernel,
        mesh=mesh,
        out_type=[jax.ShapeDtypeStruct((B, 2 * D), jnp.float32)
                  for _ in range(NT // 2)],
        scratch_types=[
            pltpu.VMEM((NT, b_per_w), jnp.int32),
            pltpu.VMEM((NT, CHUNK, D), jnp.float32),
            pltpu.SemaphoreType.DMA,
            pltpu.SemaphoreType.DMA,
            pltpu.SemaphoreType.DMA,
            pltpu.SemaphoreType.DMA,
        ],
        compiler_params=pltpu.CompilerParams(use_tc_tiling_on_sc=False),
    )
    def k(h_ref, s_ref, m_ref, hr_ref, cmod_ref, cmak_ref,
          ht_ref, st_ref, mt_ref, hrt_ref, cmodt_ref, cmakt_ref,
          o0, o1, o2, idx_v, bufs, sem_g0, sem_g1, sem_w0, sem_w1):
        wid = lax.axis_index("s") * NC + lax.axis_index("c")
        base = wid * b_per_w
        idx_hbm = [h_ref, s_ref, m_ref, hr_ref, cmod_ref, cmak_ref]
        tbls = [ht_ref, st_ref, mt_ref, hrt_ref, cmodt_ref, cmakt_ref]
        outs = [o0, o1, o2]
        sem_g = [sem_g0, sem_g1]
        sem_w = [sem_w0, sem_w1]
        idx_copies = [
            pltpu.async_copy(idx_hbm[t].at[pl.ds(base, b_per_w)],
                             idx_v.at[t], sem_g0)
            for t in range(NT)
        ]
        for d in idx_copies:
            d.wait()

        def group_tables(grp):
            return range(GRP0) if grp == 0 else range(GRP0, NT)

        def fire_task(i):
            c, grp = i // 2, i % 2
            ds = []
            for t in group_tables(grp):
                ds.append(pltpu.async_copy(
                    tbls[t].at[idx_v.at[t, pl.ds(c * CHUNK, CHUNK)]],
                    bufs.at[t], sem_g[grp]))
            return ds

        def fire_writes(i):
            c, grp = i // 2, i % 2
            off = base + c * CHUNK
            ds = []
            for t in group_tables(grp):
                ds.append(pltpu.async_copy(
                    bufs.at[t],
                    outs[t // 2].at[pl.ds(off, CHUNK),
                                    pl.ds((t % 2) * D, D)],
                    sem_w[grp]))
            return ds

        writes = {0: [], 1: []}
        gath = {0: [], 1: []}
        for i in range(n_tasks + 1):
            if i < n_tasks:
                s = i % 2
                for wdesc in writes[s]:
                    wdesc.wait()
                writes[s] = []
                gath[s] = fire_task(i)
            if i >= 1:
                sj = (i - 1) % 2
                for gd in gath[sj]:
                    gd.wait()
                gath[sj] = []
                writes[sj] = fire_writes(i - 1)
        for s in (0, 1):
            for wdesc in writes[s]:
                wdesc.wait()

    return k(h_i, s_i, m_i, hr_i, cmod_i, cmak_i,
             h_t, s_t, m_t, hr_t, cmod_t, cmak_t)


def _asm_body(p0, p1, p2, lat_ref, lon_ref,
              w1_ref, b1_ref, w2_ref, b2_ref, out_ref):
    for t, p in enumerate((p0, p1, p2)):
        out_ref[:, t * 2 * D:(t + 1) * 2 * D] = p[...]
    h = jnp.maximum(
        lat_ref[...] * w1_ref[0:1, :] + lon_ref[...] * w1_ref[1:2, :]
        + b1_ref[...],
        0.0,
    )
    out_ref[:, NT * D:] = (
        jnp.dot(h, w2_ref[...], preferred_element_type=jnp.float32)
        + b2_ref[...]
    )


def _assemble(planes, latitude, longitude, W1, b1, W2, b2):
    plane_spec = pl.BlockSpec((BLK, 2 * D), lambda i: (i, 0))
    col_spec = pl.BlockSpec((BLK, 1), lambda i: (i, 0))
    w1_spec = pl.BlockSpec((2, 32), lambda i: (0, 0))
    b1_spec = pl.BlockSpec((1, 32), lambda i: (0, 0))
    w2_spec = pl.BlockSpec((32, D), lambda i: (0, 0))
    b2_spec = pl.BlockSpec((1, D), lambda i: (0, 0))
    return pl.pallas_call(
        _asm_body,
        grid=(B // BLK,),
        in_specs=[plane_spec] * 3 + [col_spec, col_spec,
                                     w1_spec, b1_spec, w2_spec, b2_spec],
        out_specs=pl.BlockSpec((BLK, (NT + 1) * D), lambda i: (i, 0)),
        out_shape=jax.ShapeDtypeStruct((B, (NT + 1) * D), jnp.float32),
    )(*planes,
      latitude.reshape(B, 1), longitude.reshape(B, 1),
      W1, b1.reshape(1, 32), W2, b2.reshape(1, D))


def kernel(habitat, substrate, month, hour, camera_model, camera_maker,
           latitude, longitude,
           habitat_table, substrate_table, month_table, hour_table,
           camera_model_table, camera_maker_table, W1, b1, W2, b2):
    idx = [x.astype(jnp.int32) for x in
           (habitat, substrate, month, hour, camera_model, camera_maker)]
    planes = _sc_gather(*idx,
                        habitat_table, substrate_table, month_table,
                        hour_table, camera_model_table, camera_maker_table)
    return _assemble(planes, latitude, longitude, W1, b1, W2, b2)


# speedup vs baseline: 1.1547x; 1.0651x over previous
import functools

import jax
import jax.numpy as jnp
from jax import lax
from jax.experimental import pallas as pl
from jax.experimental.pallas import tpu as pltpu
from jax.experimental.pallas import tpu_sc as plsc

B = 16384
D = 64
NT = 6
CHUNK = 128
GRP0 = 3
BLK = 1024


def _sc_gather(idxs, tbls_in):
    N = len(tbls_in)
    NP = N // 2
    G0 = N // 2
    info = plsc.get_sparse_core_info()
    NC, NS = info.num_cores, info.num_subcores
    NW = NC * NS
    b_per_w = B // NW
    n_sub = b_per_w // CHUNK
    n_tasks = n_sub * 2

    mesh = plsc.VectorSubcoreMesh(core_axis_name="c", subcore_axis_name="s")

    @functools.partial(
        pl.kernel,
        mesh=mesh,
        out_type=[jax.ShapeDtypeStruct((B, 2 * D), jnp.float32)
                  for _ in range(NP)],
        scratch_types=[
            pltpu.VMEM((N, b_per_w), jnp.int32),
            pltpu.VMEM((N, CHUNK, D), jnp.float32),
            pltpu.SemaphoreType.DMA,
            pltpu.SemaphoreType.DMA,
            pltpu.SemaphoreType.DMA,
            pltpu.SemaphoreType.DMA,
        ],
        compiler_params=pltpu.CompilerParams(use_tc_tiling_on_sc=False),
    )
    def k(*refs):
        idx_hbm = list(refs[:N])
        tbls = list(refs[N:2 * N])
        outs = list(refs[2 * N:2 * N + NP])
        idx_v, bufs, sem_g0, sem_g1, sem_w0, sem_w1 = refs[2 * N + NP:]
        wid = lax.axis_index("s") * NC + lax.axis_index("c")
        base = wid * b_per_w
        sem_g = [sem_g0, sem_g1]
        sem_w = [sem_w0, sem_w1]
        idx_copies = [
            pltpu.async_copy(idx_hbm[t].at[pl.ds(base, b_per_w)],
                             idx_v.at[t], sem_g0)
            for t in range(N)
        ]
        for d in idx_copies:
            d.wait()

        def group_tables(grp):
            return range(G0) if grp == 0 else range(G0, N)

        def fire_task(i):
            c, grp = i // 2, i % 2
            ds = []
            for t in group_tables(grp):
                ds.append(pltpu.async_copy(
                    tbls[t].at[idx_v.at[t, pl.ds(c * CHUNK, CHUNK)]],
                    bufs.at[t], sem_g[grp]))
            return ds

        def fire_writes(i):
            c, grp = i // 2, i % 2
            off = base + c * CHUNK
            ds = []
            for t in group_tables(grp):
                ds.append(pltpu.async_copy(
                    bufs.at[t],
                    outs[t // 2].at[pl.ds(off, CHUNK),
                                    pl.ds((t % 2) * D, D)],
                    sem_w[grp]))
            return ds

        writes = {0: [], 1: []}
        gath = {0: [], 1: []}
        for i in range(n_tasks + 1):
            if i < n_tasks:
                s = i % 2
                for wdesc in writes[s]:
                    wdesc.wait()
                writes[s] = []
                gath[s] = fire_task(i)
            if i >= 1:
                sj = (i - 1) % 2
                for gd in gath[sj]:
                    gd.wait()
                gath[sj] = []
                writes[sj] = fire_writes(i - 1)
        for s in (0, 1):
            for wdesc in writes[s]:
                wdesc.wait()

    return k(*idxs, *tbls_in)


def _asm_body(p0, p1, p2, lat_ref, lon_ref,
              w1_ref, b1_ref, w2_ref, b2_ref, out_ref):
    for t, p in enumerate((p0, p1, p2)):
        out_ref[:, t * 2 * D:(t + 1) * 2 * D] = p[...]
    h = jnp.maximum(
        lat_ref[...] * w1_ref[0:1, :] + lon_ref[...] * w1_ref[1:2, :]
        + b1_ref[...],
        0.0,
    )
    out_ref[:, NT * D:] = (
        jnp.dot(h, w2_ref[...], preferred_element_type=jnp.float32)
        + b2_ref[...]
    )


def _assemble(planes, latitude, longitude, W1, b1, W2, b2):
    plane_spec = pl.BlockSpec((BLK, 2 * D), lambda i: (i, 0))
    col_spec = pl.BlockSpec((BLK, 1), lambda i: (i, 0))
    w1_spec = pl.BlockSpec((2, 32), lambda i: (0, 0))
    b1_spec = pl.BlockSpec((1, 32), lambda i: (0, 0))
    w2_spec = pl.BlockSpec((32, D), lambda i: (0, 0))
    b2_spec = pl.BlockSpec((1, D), lambda i: (0, 0))
    return pl.pallas_call(
        _asm_body,
        grid=(B // BLK,),
        in_specs=[plane_spec] * 3 + [col_spec, col_spec,
                                     w1_spec, b1_spec, w2_spec, b2_spec],
        out_specs=pl.BlockSpec((BLK, (NT + 1) * D), lambda i: (i, 0)),
        out_shape=jax.ShapeDtypeStruct((B, (NT + 1) * D), jnp.float32),
    )(*planes,
      latitude.reshape(B, 1), longitude.reshape(B, 1),
      W1, b1.reshape(1, 32), W2, b2.reshape(1, D))


def kernel(habitat, substrate, month, hour, camera_model, camera_maker,
           latitude, longitude,
           habitat_table, substrate_table, month_table, hour_table,
           camera_model_table, camera_maker_table, W1, b1, W2, b2):
    idx = [x.astype(jnp.int32) for x in
           (habitat, substrate, month, hour, camera_model, camera_maker)]
    planes_small = _sc_gather(
        idx[:4], [habitat_table, substrate_table, month_table, hour_table])
    planes_cam = _sc_gather(
        idx[4:], [camera_model_table, camera_maker_table])
    return _assemble(planes_small + planes_cam, latitude, longitude,
                     W1, b1, W2, b2)


# speedup vs baseline: 1.6406x; 1.4208x over previous
import functools

import jax
import jax.numpy as jnp
from jax import lax
from jax.experimental import pallas as pl
from jax.experimental.pallas import tpu as pltpu
from jax.experimental.pallas import tpu_sc as plsc

B = 16384
D = 64
NT = 6
CHUNK = 128
GRP0 = 3
BLK = 1024


def _sc_gather(idxs, tbls_in):
    N = len(tbls_in)
    NP = N // 2
    G0 = N // 2
    info = plsc.get_sparse_core_info()
    NC, NS = info.num_cores, info.num_subcores
    NW = NC * NS
    b_per_w = B // NW
    n_sub = b_per_w // CHUNK
    n_tasks = n_sub * 2

    mesh = plsc.VectorSubcoreMesh(core_axis_name="c", subcore_axis_name="s")

    @functools.partial(
        pl.kernel,
        mesh=mesh,
        out_type=[jax.ShapeDtypeStruct((B, 2 * D), jnp.float32)
                  for _ in range(NP)],
        scratch_types=[
            pltpu.VMEM((N, b_per_w), jnp.int32),
            pltpu.VMEM((N, CHUNK, D), jnp.float32),
            pltpu.SemaphoreType.DMA,
            pltpu.SemaphoreType.DMA,
            pltpu.SemaphoreType.DMA,
            pltpu.SemaphoreType.DMA,
        ],
        compiler_params=pltpu.CompilerParams(use_tc_tiling_on_sc=False),
    )
    def k(*refs):
        idx_hbm = list(refs[:N])
        tbls = list(refs[N:2 * N])
        outs = list(refs[2 * N:2 * N + NP])
        idx_v, bufs, sem_g0, sem_g1, sem_w0, sem_w1 = refs[2 * N + NP:]
        wid = lax.axis_index("s") * NC + lax.axis_index("c")
        base = wid * b_per_w
        sem_g = [sem_g0, sem_g1]
        sem_w = [sem_w0, sem_w1]
        idx_copies = [
            pltpu.async_copy(idx_hbm[t].at[pl.ds(base, b_per_w)],
                             idx_v.at[t], sem_g0)
            for t in range(N)
        ]
        for d in idx_copies:
            d.wait()

        def group_tables(grp):
            return range(G0) if grp == 0 else range(G0, N)

        def fire_task(i):
            c, grp = i // 2, i % 2
            ds = []
            for t in group_tables(grp):
                ds.append(pltpu.async_copy(
                    tbls[t].at[idx_v.at[t, pl.ds(c * CHUNK, CHUNK)]],
                    bufs.at[t], sem_g[grp]))
            return ds

        def fire_writes(i):
            c, grp = i // 2, i % 2
            off = base + c * CHUNK
            ds = []
            for t in group_tables(grp):
                ds.append(pltpu.async_copy(
                    bufs.at[t],
                    outs[t // 2].at[pl.ds(off, CHUNK),
                                    pl.ds((t % 2) * D, D)],
                    sem_w[grp]))
            return ds

        writes = {0: [], 1: []}
        gath = {0: [], 1: []}
        for i in range(n_tasks + 1):
            if i < n_tasks:
                s = i % 2
                for wdesc in writes[s]:
                    wdesc.wait()
                writes[s] = []
                gath[s] = fire_task(i)
            if i >= 1:
                sj = (i - 1) % 2
                for gd in gath[sj]:
                    gd.wait()
                gath[sj] = []
                writes[sj] = fire_writes(i - 1)
        for s in (0, 1):
            for wdesc in writes[s]:
                wdesc.wait()

    return k(*idxs, *tbls_in)


def _sc_gather_small(idxs, hab_t, sub_t, mon_t, hr_t):
    info = plsc.get_sparse_core_info()
    NC, NS = info.num_cores, info.num_subcores
    NW = NC * NS
    b_per_w = B // NW
    n_sub = b_per_w // CHUNK
    n_tasks = n_sub * 2

    mesh = plsc.VectorSubcoreMesh(core_axis_name="c", subcore_axis_name="s")

    @functools.partial(
        pl.kernel,
        mesh=mesh,
        out_type=[jax.ShapeDtypeStruct((B, 2 * D), jnp.float32)
                  for _ in range(2)],
        scratch_types=[
            pltpu.VMEM((4, b_per_w), jnp.int32),
            pltpu.VMEM((2, CHUNK, D), jnp.float32),
            pltpu.VMEM((b_per_w, 2 * D), jnp.float32),
            pltpu.VMEM((12, D), jnp.float32),
            pltpu.VMEM((24, D), jnp.float32),
            pltpu.SemaphoreType.DMA,
            pltpu.SemaphoreType.DMA,
            pltpu.SemaphoreType.DMA,
            pltpu.SemaphoreType.DMA,
        ],
        compiler_params=pltpu.CompilerParams(use_tc_tiling_on_sc=False),
    )
    def k(h_ref, s_ref, m_ref, hr_ref,
          ht_ref, st_ref, mt_ref, hrt_ref,
          o01, o23, idx_v, bufs, mh_buf, mt_v, hr_v,
          sem_g0, sem_g1, sem_w0, sem_w1):
        wid = lax.axis_index("s") * NC + lax.axis_index("c")
        base = wid * b_per_w
        sem_g = [sem_g0, sem_g1]
        sem_w = [sem_w0, sem_w1]
        idx_hbm = [h_ref, s_ref, m_ref, hr_ref]
        stage = [
            pltpu.async_copy(idx_hbm[t].at[pl.ds(base, b_per_w)],
                             idx_v.at[t], sem_g0)
            for t in range(4)
        ]
        stage.append(pltpu.async_copy(mt_ref, mt_v, sem_g0))
        stage.append(pltpu.async_copy(hrt_ref, hr_v, sem_g0))
        for d in stage:
            d.wait()

        def fire_task(i):
            c, t = i // 2, i % 2
            tref = ht_ref if t == 0 else st_ref
            return [pltpu.async_copy(
                tref.at[idx_v.at[t, pl.ds(c * CHUNK, CHUNK)]],
                bufs.at[t], sem_g[t])]

        def fire_writes(i):
            c, t = i // 2, i % 2
            off = base + c * CHUNK
            return [pltpu.async_copy(
                bufs.at[t],
                o01.at[pl.ds(off, CHUNK), pl.ds(t * D, D)],
                sem_w[t])]

        writes = {0: [], 1: []}
        gath = {0: [], 1: []}
        mh_done = [False]
        for i in range(n_tasks + 1):
            if i < n_tasks:
                s = i % 2
                for wdesc in writes[s]:
                    wdesc.wait()
                writes[s] = []
                gath[s] = fire_task(i)
            if not mh_done[0]:
                mh_done[0] = True

                @pl.loop(0, b_per_w // 16)
                def _(g):
                    mv = idx_v[2, pl.ds(g * 16, 16)]
                    hv = idx_v[3, pl.ds(g * 16, 16)]
                    for r in range(16):
                        j = g * 16 + r
                        m = mv[r]
                        h = hv[r]
                        for q in range(D // 16):
                            sl = pl.ds(q * 16, 16)
                            mh_buf[j, sl] = mt_v[m, sl]
                            mh_buf[j, pl.ds(D + q * 16, 16)] = hr_v[h, sl]
            if i >= 1:
                sj = (i - 1) % 2
                for gd in gath[sj]:
                    gd.wait()
                gath[sj] = []
                writes[sj] = fire_writes(i - 1)
        pltpu.sync_copy(mh_buf, o23.at[pl.ds(base, b_per_w)])
        for s in (0, 1):
            for wdesc in writes[s]:
                wdesc.wait()

    return k(*idxs, hab_t, sub_t, mon_t, hr_t)


def _asm_body(p0, p1, p2, lat_ref, lon_ref,
              w1_ref, b1_ref, w2_ref, b2_ref, out_ref):
    for t, p in enumerate((p0, p1, p2)):
        out_ref[:, t * 2 * D:(t + 1) * 2 * D] = p[...]
    h = jnp.maximum(
        lat_ref[...] * w1_ref[0:1, :] + lon_ref[...] * w1_ref[1:2, :]
        + b1_ref[...],
        0.0,
    )
    out_ref[:, NT * D:] = (
        jnp.dot(h, w2_ref[...], preferred_element_type=jnp.float32)
        + b2_ref[...]
    )


def _assemble(planes, latitude, longitude, W1, b1, W2, b2):
    plane_spec = pl.BlockSpec((BLK, 2 * D), lambda i: (i, 0))
    col_spec = pl.BlockSpec((BLK, 1), lambda i: (i, 0))
    w1_spec = pl.BlockSpec((2, 32), lambda i: (0, 0))
    b1_spec = pl.BlockSpec((1, 32), lambda i: (0, 0))
    w2_spec = pl.BlockSpec((32, D), lambda i: (0, 0))
    b2_spec = pl.BlockSpec((1, D), lambda i: (0, 0))
    return pl.pallas_call(
        _asm_body,
        grid=(B // BLK,),
        in_specs=[plane_spec] * 3 + [col_spec, col_spec,
                                     w1_spec, b1_spec, w2_spec, b2_spec],
        out_specs=pl.BlockSpec((BLK, (NT + 1) * D), lambda i: (i, 0)),
        out_shape=jax.ShapeDtypeStruct((B, (NT + 1) * D), jnp.float32),
    )(*planes,
      latitude.reshape(B, 1), longitude.reshape(B, 1),
      W1, b1.reshape(1, 32), W2, b2.reshape(1, D))


def kernel(habitat, substrate, month, hour, camera_model, camera_maker,
           latitude, longitude,
           habitat_table, substrate_table, month_table, hour_table,
           camera_model_table, camera_maker_table, W1, b1, W2, b2):
    idx = [x.astype(jnp.int32) for x in
           (habitat, substrate, month, hour, camera_model, camera_maker)]
    planes_small = _sc_gather_small(
        idx[:4], habitat_table, substrate_table, month_table, hour_table)
    planes_cam = _sc_gather(
        idx[4:], [camera_model_table, camera_maker_table])
    return _assemble(planes_small + planes_cam, latitude, longitude,
                     W1, b1, W2, b2)
